# baseline (speedup 1.0000x reference)
import jax
import jax.numpy as jnp
from jax.experimental import pallas as pl
from jax.experimental.pallas import tpu as pltpu

B2, L, H, D = 16, 2048, 16, 128
HD = H * D
CW = 32
NBUF = 8
RA = 4
NCH = L // CW


def _relay_body(idx_ref, ck, cv, k_ref, v_ref, ok, ov, buf, sem_r, sem_w,
                sem_s):
    idx = idx_ref[0]
    chunks = []
    for i in range(NCH):
        chunks.append((ck, ok, k_ref, i))
    for i in range(NCH):
        chunks.append((cv, ov, v_ref, i))

    def rd(c, slot):
        src, _, _, i = c
        return pltpu.make_async_copy(
            src.at[:, pl.ds(i * CW, CW), :], buf.at[slot], sem_r.at[slot])

    def wr(c, slot):
        _, dst, _, i = c
        return pltpu.make_async_copy(
            buf.at[slot], dst.at[:, pl.ds(i * CW, CW), :], sem_w.at[slot])

    for j in range(min(RA, len(chunks))):
        rd(chunks[j], j % NBUF).start()
    for j, c in enumerate(chunks):
        slot = j % NBUF
        rd(c, slot).wait()
        _, _, new_ref, i = c
        r = idx - i * CW
        @pl.when(jnp.logical_and(r >= 0, r < CW))
        def _(slot=slot, new_ref=new_ref, r=r):
            cp = pltpu.make_async_copy(
                new_ref, buf.at[slot, :, pl.ds(r, 1), :], sem_s)
            cp.start()
            cp.wait()
        wr(c, slot).start()
        nxt = j + RA
        if nxt < len(chunks):
            prev = nxt - NBUF
            if prev >= 0:
                wr(chunks[prev], prev % NBUF).wait()
            rd(chunks[nxt], nxt % NBUF).start()
    for j in range(max(0, len(chunks) - NBUF), len(chunks)):
        wr(chunks[j], j % NBUF).wait()


def kernel(cache_k, cache_v, k, v, current_idx):
    ck = cache_k.reshape(B2, L, HD)
    cv = cache_v.reshape(B2, L, HD)
    k3 = k.reshape(B2, 1, HD)
    v3 = v.reshape(B2, 1, HD)
    idx = jnp.asarray(current_idx, jnp.int32).reshape(1)

    ok, ov = pl.pallas_call(
        _relay_body,
        in_specs=[
            pl.BlockSpec(memory_space=pltpu.MemorySpace.SMEM),
            pl.BlockSpec(memory_space=pltpu.MemorySpace.HBM),
            pl.BlockSpec(memory_space=pltpu.MemorySpace.HBM),
            pl.BlockSpec(memory_space=pltpu.MemorySpace.HBM),
            pl.BlockSpec(memory_space=pltpu.MemorySpace.HBM),
        ],
        out_specs=[
            pl.BlockSpec(memory_space=pltpu.MemorySpace.HBM),
            pl.BlockSpec(memory_space=pltpu.MemorySpace.HBM),
        ],
        out_shape=[
            jax.ShapeDtypeStruct((B2, L, HD), jnp.float32),
            jax.ShapeDtypeStruct((B2, L, HD), jnp.float32),
        ],
        scratch_shapes=[
            pltpu.VMEM((NBUF, B2, CW, HD), jnp.float32),
            pltpu.SemaphoreType.DMA((NBUF,)),
            pltpu.SemaphoreType.DMA((NBUF,)),
            pltpu.SemaphoreType.DMA,
        ],
    )(idx, ck, cv, k3, v3)
    return ok.reshape(B2, L, H, D), ov.reshape(B2, L, H, D)


# speedup vs baseline: 1.1095x; 1.1095x over previous
import functools
import jax
import jax.numpy as jnp
from jax import lax
from jax.experimental import pallas as pl
from jax.experimental.pallas import tpu as pltpu
from jax.experimental.pallas import tpu_sc as plsc

B2, L, H, D = 16, 2048, 16, 128
HD = H * D
CRS = 16
NB = 2
NCHT = L // CRS
NG = NCHT // NB


def _sc_body(ck, cv, k3, v3, idxh, ok, ov, buf, idxv, sem_r, sem_w):
    info = plsc.get_sparse_core_info()
    nc = info.num_cores
    wid = lax.axis_index("s") * nc + lax.axis_index("c")
    cache_sel = wid % 2
    batch = wid // 2

    pltpu.sync_copy(idxh, idxv)
    idx = idxv[...][0]

    def run(src, dst, row_src):
        base = batch * L

        def rd(c, slot):
            return pltpu.make_async_copy(
                src.at[pl.ds(base + c * CRS, CRS), :], buf.at[slot],
                sem_r.at[slot])

        def wr(c, slot):
            return pltpu.make_async_copy(
                buf.at[slot], dst.at[pl.ds(base + c * CRS, CRS), :],
                sem_w.at[slot])

        for s in range(NB):
            rd(s, s).start()

        def group(g, _):
            for s in range(NB):
                c = g * NB + s
                rd(c, s).wait()
                rl = idx - c * CRS
                @pl.when(jnp.logical_and(rl >= 0, rl < CRS))
                def _(s=s, rl=rl):
                    pltpu.sync_copy(
                        row_src.at[pl.ds(batch, 1), :],
                        buf.at[s, pl.ds(rl, 1), :])
                wr(c, s).start()
            for s in range(NB):
                @pl.when(g + 1 < NG)
                def _(g=g, s=s):
                    wr(g * NB + s, s).wait()
                    rd((g + 1) * NB + s, s).start()
            return 0

        lax.fori_loop(0, NG, group, 0)
        for s in range(NB):
            wr((NG - 1) * NB + s, s).wait()

    @pl.when(cache_sel == 0)
    def _():
        run(ck, ok, k3)

    @pl.when(cache_sel == 1)
    def _():
        run(cv, ov, v3)


def kernel(cache_k, cache_v, k, v, current_idx):
    ck = cache_k.reshape(B2 * L, HD)
    cv = cache_v.reshape(B2 * L, HD)
    k3 = k.reshape(B2, HD)
    v3 = v.reshape(B2, HD)
    idx = jnp.full((16,), current_idx, jnp.int32)

    mesh = plsc.VectorSubcoreMesh(core_axis_name="c", subcore_axis_name="s")
    f = functools.partial(
        pl.kernel,
        mesh=mesh,
        out_type=[
            jax.ShapeDtypeStruct((B2 * L, HD), jnp.float32),
            jax.ShapeDtypeStruct((B2 * L, HD), jnp.float32),
        ],
        scratch_types=[
            pltpu.VMEM((NB, CRS, HD), jnp.float32),
            pltpu.VMEM((16,), jnp.int32),
            pltpu.SemaphoreType.DMA((NB,)),
            pltpu.SemaphoreType.DMA((NB,)),
        ],
    )(_sc_body)
    ok, ov = f(ck, cv, k3, v3, idx)
    return ok.reshape(B2, L, H, D), ov.reshape(B2, L, H, D)


# speedup vs baseline: 1.3597x; 1.2255x over previous
import jax
import jax.numpy as jnp
from jax.experimental import pallas as pl
from jax.experimental.pallas import tpu as pltpu

B2, L, H, D = 16, 2048, 16, 128
HD = H * D
W = 8


def _scatter_body(idx_ref, k_ref, v_ref, ck_ref, cv_ref, ok_ref, ov_ref):
    r = idx_ref[0] % W
    ok_ref[...] = ck_ref[...]
    ov_ref[...] = cv_ref[...]
    ok_ref[:, pl.ds(r, 1), :] = k_ref[...][:, None, :]
    ov_ref[:, pl.ds(r, 1), :] = v_ref[...][:, None, :]


def kernel(cache_k, cache_v, k, v, current_idx):
    ck = cache_k.reshape(B2, L, HD)
    cv = cache_v.reshape(B2, L, HD)
    k2 = k.reshape(B2, HD)
    v2 = v.reshape(B2, HD)
    idx = jnp.asarray(current_idx, jnp.int32).reshape(1)

    win = lambda i, idx_ref: (0, idx_ref[0] // W, 0)
    grid_spec = pltpu.PrefetchScalarGridSpec(
        num_scalar_prefetch=1,
        grid=(1,),
        in_specs=[
            pl.BlockSpec((B2, HD), lambda i, idx_ref: (0, 0)),
            pl.BlockSpec((B2, HD), lambda i, idx_ref: (0, 0)),
            pl.BlockSpec((B2, W, HD), win),
            pl.BlockSpec((B2, W, HD), win),
        ],
        out_specs=[
            pl.BlockSpec((B2, W, HD), win),
            pl.BlockSpec((B2, W, HD), win),
        ],
    )

    ok, ov = pl.pallas_call(
        _scatter_body,
        grid_spec=grid_spec,
        out_shape=[
            jax.ShapeDtypeStruct((B2, L, HD), jnp.float32),
            jax.ShapeDtypeStruct((B2, L, HD), jnp.float32),
        ],
        input_output_aliases={3: 0, 4: 1},
    )(idx, k2, v2, ck, cv)
    return ok.reshape(B2, L, H, D), ov.reshape(B2, L, H, D)
